# baseline (speedup 1.0000x reference)
import functools

import jax
import jax.numpy as jnp
from jax import lax
from jax.experimental import pallas as pl
from jax.experimental.pallas import tpu as pltpu
from jax.experimental.pallas import tpu_sc as plsc

NUM_CORES = 2
NUM_SUBCORES = 16
NUM_WORKERS = NUM_CORES * NUM_SUBCORES

BATCH = 16384
HIST_LEN = 50
DIM = 64
VOCAB = 1000000

BCOLS = BATCH // 128
BCOLS_PER_W = BCOLS // NUM_WORKERS
N_BLOCKS2 = HIST_LEN * BCOLS_PER_W
N_GROUPS2 = N_BLOCKS2 // 2

_MESH = plsc.VectorSubcoreMesh(
    core_axis_name="c",
    subcore_axis_name="s",
    num_cores=NUM_CORES,
    num_subcores=NUM_SUBCORES,
)

_PARAMS = pltpu.CompilerParams(use_tc_tiling_on_sc=True,
                               needs_layout_passes=False)


def _lane():
    return lax.iota(jnp.int32, 16)


@functools.partial(
    pl.kernel,
    out_type=jax.ShapeDtypeStruct((HIST_LEN, DIM, BATCH), jnp.float32),
    mesh=_MESH,
    scratch_types=(
        [pltpu.VMEM((HIST_LEN, 512), jnp.int32)]
        + [pltpu.VMEM((128,), jnp.int32) for _ in range(2)]
        + [pltpu.VMEM((128, 128), jnp.float32) for _ in range(2)]
        + [pltpu.VMEM((1, DIM, 128), jnp.float32) for _ in range(2)]
        + [pltpu.SemaphoreType.DMA for _ in range(4)]
    ),
    compiler_params=_PARAMS,
)
def _gather_kernel(packed, x_t, out_hbm, idx_slab, pi0, pi1, rw0, rw1,
                   ot0, ot1, sg0, sg1, sw0, sw1):
    wid = lax.axis_index("s") * NUM_CORES + lax.axis_index("c")
    pidx = [pi0, pi1]
    rows = [rw0, rw1]
    outs = [ot0, ot1]
    sem_g = [sg0, sg1]
    sem_w = [sw0, sw1]
    b0 = wid * 512

    pltpu.sync_copy(x_t.at[:, pl.ds(pl.multiple_of(b0, 512), 512)],
                    idx_slab)

    def t_of(k):
        return k // BCOLS_PER_W

    def c_of(k):
        return lax.rem(k, BCOLS_PER_W)

    def start_gather(k, b):
        t = t_of(k)
        c = c_of(k)
        for l16 in range(8):
            r = idx_slab[t, pl.ds(c * 128 + l16 * 16, 16)]
            pidx[b][pl.ds(l16 * 16, 16)] = r
        pltpu.async_copy(packed.at[pidx[b]], rows[b], sem_g[b])

    def gather_wait(b):
        pltpu.make_async_copy(packed.at[pidx[b]], rows[b], sem_g[b]).wait()

    def out_desc(k, b):
        t = t_of(k)
        c = c_of(k)
        return pltpu.make_async_copy(
            outs[b],
            out_hbm.at[pl.ds(t, 1), :,
                       pl.ds(pl.multiple_of(b0 + c * 128, 128), 128)],
            sem_w[b])

    def transpose_block(b):
        ov = outs[b].at[0]
        rows_static = [_lane() + 16 * l16 for l16 in range(8)]

        @plsc.parallel_loop(0, DIM, unroll=4)
        def _(d):
            col_d = jnp.broadcast_to(d, (16,))
            vs = [plsc.load_gather(rows[b], [rows_static[l16], col_d])
                  for l16 in range(8)]
            for l16 in range(8):
                ov[d, pl.ds(l16 * 16, 16)] = vs[l16]

    start_gather(0, 0)

    def group(g, _):
        for b in range(2):
            k = 2 * g + b

            @pl.when(k + 1 < N_BLOCKS2)
            def _():
                start_gather(k + 1, 1 - b)

            gather_wait(b)

            @pl.when(k >= 2)
            def _():
                out_desc(k - 2, b).wait()

            transpose_block(b)
            out_desc(k, b).start()
        return 0

    lax.fori_loop(0, N_GROUPS2, group, 0)

    for b in range(2):
        out_desc(N_BLOCKS2 - 2 + b, b).wait()


def kernel(x, embedding):
    padded = jnp.pad(embedding, ((0, 0), (0, 128 - DIM)))
    out_t = _gather_kernel(padded, x.T)
    return out_t.transpose(2, 0, 1)


# speedup vs baseline: 1.8887x; 1.8887x over previous
import functools

import jax
import jax.numpy as jnp
from jax import lax
from jax.experimental import pallas as pl
from jax.experimental.pallas import tpu as pltpu
from jax.experimental.pallas import tpu_sc as plsc

NUM_CORES = 2
NUM_SUBCORES = 16
NUM_WORKERS = NUM_CORES * NUM_SUBCORES

BATCH = 16384
HIST_LEN = 50
DIM = 64
VOCAB = 1000000
ROWS_PER_W = BATCH // NUM_WORKERS
NBUF = 8
N_GROUPS = ROWS_PER_W // NBUF

_MESH = plsc.VectorSubcoreMesh(
    core_axis_name="c",
    subcore_axis_name="s",
    num_cores=NUM_CORES,
    num_subcores=NUM_SUBCORES,
)


@functools.partial(
    pl.kernel,
    out_type=jax.ShapeDtypeStruct((BATCH, HIST_LEN, DIM), jnp.float32),
    mesh=_MESH,
    scratch_types=(
        [pltpu.VMEM((ROWS_PER_W, HIST_LEN), jnp.int32)]
        + [pltpu.VMEM((1, HIST_LEN, DIM), jnp.float32) for _ in range(NBUF)]
        + [pltpu.SemaphoreType.DMA for _ in range(2 * NBUF)]
    ),
    compiler_params=pltpu.CompilerParams(use_tc_tiling_on_sc=False),
)
def _gather_kernel(table_hbm, idx_hbm, out_hbm, idx_all, *bufs):
    rows = list(bufs[:NBUF])
    sem_g = list(bufs[NBUF:2 * NBUF])
    sem_w = list(bufs[2 * NBUF:])

    wid = lax.axis_index("s") * NUM_CORES + lax.axis_index("c")
    base = wid * ROWS_PER_W

    pltpu.sync_copy(idx_hbm.at[pl.ds(base, ROWS_PER_W)], idx_all)

    def gather_desc(j, b):
        return pltpu.make_async_copy(
            table_hbm.at[idx_all.at[pl.ds(j, 1)]], rows[b], sem_g[b])

    def write_desc(j, b):
        return pltpu.make_async_copy(
            rows[b], out_hbm.at[pl.ds(base + j, 1)], sem_w[b])

    for b in range(NBUF - 1):
        gather_desc(b, b).start()

    def group(g, _):
        for b in range(NBUF):
            j = g * NBUF + b
            bn = (b + NBUF - 1) % NBUF
            jn = j + NBUF - 1

            @pl.when(jn < ROWS_PER_W)
            def _():
                @pl.when(jn >= NBUF)
                def _():
                    write_desc(jn - NBUF, bn).wait()

                gather_desc(jn, bn).start()

            gather_desc(j, b).wait()
            write_desc(j, b).start()
        return 0

    lax.fori_loop(0, N_GROUPS, group, 0)

    for b in range(NBUF):
        write_desc(ROWS_PER_W - NBUF + b, b).wait()


def kernel(x, embedding):
    return _gather_kernel(embedding.reshape(1, VOCAB, DIM), x)
